# baseline (speedup 1.0000x reference)
import jax
import jax.numpy as jnp
from jax import lax
from jax.experimental import pallas as pl
from jax.experimental.pallas import tpu as pltpu

B = 16
S = 32768
D_IN = 64
D_SLOT = 64
K = 16
CHUNK = 16384
NCHUNK = S // CHUNK
SCALE = 1.0 / (D_SLOT ** 0.5)
NEG = -1e30

ROWS = S // 128


def _pass1_body(x_ref, mask_ref, w1_ref, b1_ref, w2_ref, b2_ref, q_ref,
                ws_ref, ctx_ref, stats_ref, m_acc, l_acc, ctx_acc):
    c = pl.program_id(1)

    @pl.when(c == 0)
    def _init():
        m_acc[...] = jnp.full((1, 1), -3e38, jnp.float32)
        l_acc[...] = jnp.zeros((1, 1), jnp.float32)
        ctx_acc[...] = jnp.zeros((1, D_SLOT), jnp.float32)

    x = x_ref[0]
    h1 = jnp.maximum(
        jnp.dot(x, w1_ref[...], preferred_element_type=jnp.float32)
        + b1_ref[0], 0.0)
    h = (jnp.dot(h1, w2_ref[...], preferred_element_type=jnp.float32)
         + b2_ref[0])
    sh = jax.lax.dot_general(
        q_ref[...], h, (((1,), (1,)), ((), ())),
        preferred_element_type=jnp.float32)
    s = (sh[0:1] * SCALE + sh[1:2] * SCALE) * 0.5
    valid = mask_ref[0] > 0.5
    ws = jnp.where(valid, s, NEG)
    ws_ref[0] = ws

    m_prev = m_acc[...]
    cm = jnp.max(ws).reshape(1, 1)
    m_new = jnp.maximum(m_prev, cm)
    alpha = jnp.exp(m_prev - m_new)
    p = jnp.exp(ws - m_new[0, 0])
    csum = jnp.sum(p).reshape(1, 1)
    ctx_acc[...] = ctx_acc[...] * alpha + jnp.dot(
        p, h, preferred_element_type=jnp.float32)
    l_acc[...] = l_acc[...] * alpha + csum
    m_acc[...] = m_new

    ctx_ref[0] = ctx_acc[...] / l_acc[...]
    lane = lax.broadcasted_iota(jnp.int32, (1, 128), 1)
    stats_ref[0] = (jnp.where(lane == 0, m_acc[0, 0], 0.0)
                    + jnp.where(lane == 1, l_acc[0, 0], 0.0))


def _pass2_body(ws_ref, stats_ref, attn_ref, gidx_ref):
    w = ws_ref[...]
    m = stats_ref[:, :, 0:1]
    l = stats_ref[:, :, 1:2]
    attn_ref[...] = jnp.exp(w - m) / l

    r = lax.broadcasted_iota(jnp.int32, (B, ROWS, 128), 1)
    c = lax.broadcasted_iota(jnp.int32, (B, ROWS, 128), 2)
    flat = r * 128 + c
    kio = lax.broadcasted_iota(jnp.int32, (B, 1, K), 2)
    boff = lax.broadcasted_iota(jnp.int32, (B, 1, K), 0) * S
    work = w
    idxv = jnp.zeros((B, 1, K), jnp.int32)
    for k in range(K):
        mk = jnp.max(jnp.max(work, axis=2, keepdims=True),
                     axis=1, keepdims=True)
        cand = jnp.where(work == mk, flat, S)
        ik = jnp.min(jnp.min(cand, axis=2, keepdims=True),
                     axis=1, keepdims=True)
        idxv = jnp.where(kio == k, ik, idxv)
        work = jnp.where(flat == ik, -jnp.inf, work)
    gidx_ref[...] = idxv + boff


def _pass3_body(gidx_ref, table_ref, w1_ref, b1_ref, w2_ref, b2_ref,
                sel_ref, xg, sems):
    NSEM = 8
    for j in range(B * K):
        pltpu.make_async_copy(
            table_ref.at[pl.ds(gidx_ref[j], 1), :],
            xg.at[pl.ds(j, 1), :], sems.at[j % NSEM]).start()
    for j in range(B * K):
        pltpu.make_async_copy(
            table_ref.at[pl.ds(gidx_ref[j], 1), :],
            xg.at[pl.ds(j, 1), :], sems.at[j % NSEM]).wait()
    h1 = jnp.maximum(
        jnp.dot(xg[...], w1_ref[...], preferred_element_type=jnp.float32)
        + b1_ref[0], 0.0)
    sel_ref[...] = (jnp.dot(h1, w2_ref[...], preferred_element_type=jnp.float32)
                    + b2_ref[0])


def kernel(slot_feats, slot_mask, W1, b1, W2, b2, q):
    b1r = b1.reshape(1, D_SLOT)
    b2r = b2.reshape(1, D_SLOT)
    mask3 = slot_mask.reshape(B, 1, S)

    ws, ctx, stats = pl.pallas_call(
        _pass1_body,
        grid=(B, NCHUNK),
        in_specs=[
            pl.BlockSpec((1, CHUNK, D_IN), lambda b, c: (b, c, 0)),
            pl.BlockSpec((1, 1, CHUNK), lambda b, c: (b, 0, c)),
            pl.BlockSpec((D_IN, D_SLOT), lambda b, c: (0, 0)),
            pl.BlockSpec((1, D_SLOT), lambda b, c: (0, 0)),
            pl.BlockSpec((D_SLOT, D_SLOT), lambda b, c: (0, 0)),
            pl.BlockSpec((1, D_SLOT), lambda b, c: (0, 0)),
            pl.BlockSpec((2, D_SLOT), lambda b, c: (0, 0)),
        ],
        out_specs=[
            pl.BlockSpec((1, 1, CHUNK), lambda b, c: (b, 0, c)),
            pl.BlockSpec((1, 1, D_SLOT), lambda b, c: (b, 0, 0)),
            pl.BlockSpec((1, 1, 128), lambda b, c: (b, 0, 0)),
        ],
        out_shape=[
            jax.ShapeDtypeStruct((B, 1, S), jnp.float32),
            jax.ShapeDtypeStruct((B, 1, D_SLOT), jnp.float32),
            jax.ShapeDtypeStruct((B, 1, 128), jnp.float32),
        ],
        scratch_shapes=[
            pltpu.VMEM((1, 1), jnp.float32),
            pltpu.VMEM((1, 1), jnp.float32),
            pltpu.VMEM((1, D_SLOT), jnp.float32),
        ],
        compiler_params=pltpu.CompilerParams(
            dimension_semantics=("arbitrary", "arbitrary")),
    )(slot_feats, mask3, W1, b1r, W2, b2r, q)

    ws3 = ws.reshape(B, ROWS, 128)
    attn3, gidx = pl.pallas_call(
        _pass2_body,
        in_specs=[
            pl.BlockSpec((B, ROWS, 128), lambda: (0, 0, 0)),
            pl.BlockSpec((B, 1, 128), lambda: (0, 0, 0)),
        ],
        out_specs=[
            pl.BlockSpec((B, ROWS, 128), lambda: (0, 0, 0)),
            pl.BlockSpec((B, 1, K), lambda: (0, 0, 0)),
        ],
        out_shape=[
            jax.ShapeDtypeStruct((B, ROWS, 128), jnp.float32),
            jax.ShapeDtypeStruct((B, 1, K), jnp.int32),
        ],
    )(ws3, stats)
    attnW = attn3.reshape(B, S)

    table = slot_feats.reshape(B * S, D_IN)
    sel = pl.pallas_call(
        _pass3_body,
        in_specs=[
            pl.BlockSpec(memory_space=pltpu.MemorySpace.SMEM),
            pl.BlockSpec(memory_space=pl.ANY),
            pl.BlockSpec((D_IN, D_SLOT), lambda: (0, 0)),
            pl.BlockSpec((1, D_SLOT), lambda: (0, 0)),
            pl.BlockSpec((D_SLOT, D_SLOT), lambda: (0, 0)),
            pl.BlockSpec((1, D_SLOT), lambda: (0, 0)),
        ],
        out_specs=pl.BlockSpec((B * K, D_SLOT), lambda: (0, 0)),
        out_shape=jax.ShapeDtypeStruct((B * K, D_SLOT), jnp.float32),
        scratch_shapes=[
            pltpu.VMEM((B * K, D_IN), jnp.float32),
            pltpu.SemaphoreType.DMA((8,)),
        ],
    )(gidx.reshape(B * K), table, W1, b1r, W2, b2r)

    return sel.reshape(B, K, D_SLOT), ctx.reshape(B, D_SLOT), attnW
